# Initial kernel scaffold; baseline (speedup 1.0000x reference)
import jax
import jax.numpy as jnp
from jax.experimental import pallas as pl


def kernel(x, edge_index, W_appnp, W_gat, a_self, a_neigh):
    raise NotImplementedError("write your pallas kernel here")


# speedup vs baseline: 15.6162x; 15.6162x over previous
import functools

import jax
import jax.numpy as jnp
from jax import lax
from jax.experimental import pallas as pl
from jax.experimental.pallas import tpu as pltpu
from jax.experimental.pallas import tpu_sc as plsc

N = 10000
E = 320000
D = 128
NC = 2
NS = 16
NW = NC * NS
NPAD = 10240
RPS = NPAD // NS
EB = 128
NU = E // EB
F32 = jnp.float32

_MESH = plsc.VectorSubcoreMesh(core_axis_name="c", subcore_axis_name="s")
_SC_PARAMS = pltpu.CompilerParams(needs_layout_passes=False)

_Z16 = lambda: jnp.zeros((16,), F32)


def _zero_shared_rows(ztile_v, sh_ref, sid):
    for b in range(RPS // 16):
        pltpu.sync_copy(ztile_v, sh_ref.at[pl.ds(sid * RPS + b * 16, 16)])


def _dump_shared_rows(sh_ref, out_ref, sid):
    pltpu.sync_copy(sh_ref.at[pl.ds(sid * RPS, RPS)],
                    out_ref.at[pl.ds(sid * RPS, RPS)])


@functools.partial(
    pl.kernel,
    out_type=(jax.ShapeDtypeStruct((NPAD,), F32),
              jax.ShapeDtypeStruct((NPAD,), F32)),
    mesh=_MESH,
    compiler_params=_SC_PARAMS,
    scratch_types=[
        pltpu.VMEM((EB,), jnp.int32),
        pltpu.VMEM((EB,), F32),
        pltpu.VMEM((16,), F32),
        pltpu.VMEM_SHARED((NPAD,), F32),
    ],
)
def _deg_kernel(src_hbm, dst_hbm, degin_hbm, degout_hbm,
                idx_v, ones_v, z16_v, deg_sh):
    cid = lax.axis_index("c")
    sid = lax.axis_index("s")
    z16_v[...] = _Z16()
    for k in range(EB // 16):
        ones_v[pl.ds(k * 16, 16)] = jnp.ones((16,), F32)
    for b in range(RPS // 16):
        pltpu.sync_copy(z16_v, deg_sh.at[pl.ds(sid * RPS + b * 16, 16)])
    plsc.subcore_barrier()

    nu = 156 + jnp.where(sid < NU - 156 * NS, 1, 0)

    def tally(edge_hbm):
        def body(u, _):
            r = u * NS + sid
            pltpu.sync_copy(edge_hbm.at[pl.ds(r * EB, EB)], idx_v)
            pltpu.sync_copy(ones_v, deg_sh.at[idx_v], add=True)
            return 0
        lax.fori_loop(0, nu, body, 0)

    @pl.when(cid == 0)
    def _():
        tally(dst_hbm)

    @pl.when(cid == 1)
    def _():
        tally(src_hbm)

    plsc.subcore_barrier()

    @pl.when(cid == 0)
    def _():
        _dump_shared_rows(deg_sh, degin_hbm, sid)

    @pl.when(cid == 1)
    def _():
        _dump_shared_rows(deg_sh, degout_hbm, sid)


@functools.partial(
    pl.kernel,
    out_type=(jax.ShapeDtypeStruct((NPAD, D), F32),
              jax.ShapeDtypeStruct((NPAD, D), F32)),
    mesh=_MESH,
    compiler_params=_SC_PARAMS,
    scratch_types=[
        pltpu.VMEM((EB,), jnp.int32),
        pltpu.VMEM((EB,), jnp.int32),
        pltpu.VMEM((EB, D), F32),
        pltpu.VMEM((16, 128), F32),
        pltpu.VMEM_SHARED((NPAD, D), F32),
        pltpu.SemaphoreType.DMA,
    ],
)
def _appnp_pass(g_hbm, src_hbm, dst_hbm, acc0_hbm, acc1_hbm,
                sidx_v, didx_v, rows_v, ztile_v, acc_sh, sem):
    cid = lax.axis_index("c")
    sid = lax.axis_index("s")
    wid = cid * NS + sid
    for r in range(16):
        for k in range(8):
            ztile_v[r, pl.ds(k * 16, 16)] = _Z16()
    _zero_shared_rows(ztile_v, acc_sh, sid)
    plsc.subcore_barrier()

    nu = 78 + jnp.where(wid < NU - 78 * NW, 1, 0)

    def body(u, _):
        base = (u * NW + wid) * EB
        pltpu.sync_copy(src_hbm.at[pl.ds(base, EB)], sidx_v)
        pltpu.sync_copy(dst_hbm.at[pl.ds(base, EB)], didx_v)
        pltpu.async_copy(g_hbm.at[sidx_v], rows_v, sem).wait()
        pltpu.sync_copy(rows_v, acc_sh.at[didx_v], add=True)
        return 0

    lax.fori_loop(0, nu, body, 0)
    plsc.subcore_barrier()

    @pl.when(cid == 0)
    def _():
        _dump_shared_rows(acc_sh, acc0_hbm, sid)

    @pl.when(cid == 1)
    def _():
        _dump_shared_rows(acc_sh, acc1_hbm, sid)


@functools.partial(
    pl.kernel,
    out_type=(jax.ShapeDtypeStruct((NPAD, D), F32),
              jax.ShapeDtypeStruct((NPAD, D), F32),
              jax.ShapeDtypeStruct((NPAD,), F32),
              jax.ShapeDtypeStruct((NPAD,), F32)),
    mesh=_MESH,
    compiler_params=_SC_PARAMS,
    scratch_types=[
        pltpu.VMEM((EB,), jnp.int32),
        pltpu.VMEM((EB,), jnp.int32),
        pltpu.VMEM((EB, D), F32),
        pltpu.VMEM((EB,), F32),
        pltpu.VMEM((16, 128), F32),
        pltpu.VMEM((NPAD,), F32),
        pltpu.VMEM((NPAD,), F32),
        pltpu.VMEM_SHARED((NPAD, D), F32),
        pltpu.VMEM_SHARED((NPAD,), F32),
        pltpu.SemaphoreType.DMA,
    ],
)
def _gat_pass(z_hbm, asf_hbm, anb_hbm, src_hbm, dst_hbm,
              accz0_hbm, accz1_hbm, s0_hbm, s1_hbm,
              sidx_v, didx_v, rows_v, ee_v, ztile_v, asf_v, anb_v,
              accz_sh, s_sh, sem):
    cid = lax.axis_index("c")
    sid = lax.axis_index("s")
    wid = cid * NS + sid
    for r in range(16):
        for k in range(8):
            ztile_v[r, pl.ds(k * 16, 16)] = _Z16()
    _zero_shared_rows(ztile_v, accz_sh, sid)
    for b in range(RPS // 128):
        pltpu.sync_copy(ztile_v.at[0],
                        s_sh.at[pl.ds(sid * RPS + b * 128, 128)])
    pltpu.sync_copy(asf_hbm, asf_v.at[pl.ds(0, N)])
    pltpu.sync_copy(anb_hbm, anb_v.at[pl.ds(0, N)])
    plsc.subcore_barrier()

    nu = 78 + jnp.where(wid < NU - 78 * NW, 1, 0)

    def body(u, _):
        base = (u * NW + wid) * EB
        pltpu.sync_copy(src_hbm.at[pl.ds(base, EB)], sidx_v)
        pltpu.sync_copy(dst_hbm.at[pl.ds(base, EB)], didx_v)
        cp = pltpu.async_copy(z_hbm.at[sidx_v], rows_v, sem)
        for k in range(EB // 16):
            sl = pl.ds(k * 16, 16)
            ad = plsc.load_gather(asf_v, [didx_v[sl]])
            an = plsc.load_gather(anb_v, [sidx_v[sl]])
            lg = ad + an
            lg = jnp.where(lg > 0, lg, 0.2 * lg)
            ee_v[sl] = jnp.exp(lg)
        cp.wait()

        def scale_16rows(t, _):
            e16 = ee_v[pl.ds(t * 16, 16)]
            for j in range(16):
                es = e16[j]
                row = t * 16 + j
                for k in range(D // 16):
                    sl = pl.ds(k * 16, 16)
                    rows_v[row, sl] = rows_v[row, sl] * es
            return 0

        lax.fori_loop(0, EB // 16, scale_16rows, 0)
        pltpu.sync_copy(rows_v, accz_sh.at[didx_v], add=True)
        pltpu.sync_copy(ee_v, s_sh.at[didx_v], add=True)
        return 0

    lax.fori_loop(0, nu, body, 0)
    plsc.subcore_barrier()

    @pl.when(cid == 0)
    def _():
        _dump_shared_rows(accz_sh, accz0_hbm, sid)
        _dump_shared_rows(s_sh, s0_hbm, sid)

    @pl.when(cid == 1)
    def _():
        _dump_shared_rows(accz_sh, accz1_hbm, sid)
        _dump_shared_rows(s_sh, s1_hbm, sid)


BR = 2000


def _mm1_body(x_ref, w_ref, di_ref, do_ref,
              h0_ref, g0_ref, ri_ref, vi_ref, ro_ref):
    h0 = jnp.dot(x_ref[...], w_ref[...], preferred_element_type=F32)
    din = di_ref[...] + 1.0
    dout = do_ref[...] + 1.0
    ri = lax.rsqrt(din)
    ro = lax.rsqrt(dout)
    h0_ref[...] = h0
    g0_ref[...] = h0 * ro
    ri_ref[...] = ri
    vi_ref[...] = 1.0 / din
    ro_ref[...] = ro


def _mm1_call(x, W, di, do):
    return pl.pallas_call(
        _mm1_body,
        grid=(N // BR,),
        in_specs=[
            pl.BlockSpec((BR, D), lambda i: (i, 0)),
            pl.BlockSpec((D, D), lambda i: (0, 0)),
            pl.BlockSpec((BR, 1), lambda i: (i, 0)),
            pl.BlockSpec((BR, 1), lambda i: (i, 0)),
        ],
        out_specs=[
            pl.BlockSpec((BR, D), lambda i: (i, 0)),
            pl.BlockSpec((BR, D), lambda i: (i, 0)),
            pl.BlockSpec((BR, 1), lambda i: (i, 0)),
            pl.BlockSpec((BR, 1), lambda i: (i, 0)),
            pl.BlockSpec((BR, 1), lambda i: (i, 0)),
        ],
        out_shape=[
            jax.ShapeDtypeStruct((N, D), F32),
            jax.ShapeDtypeStruct((N, D), F32),
            jax.ShapeDtypeStruct((N, 1), F32),
            jax.ShapeDtypeStruct((N, 1), F32),
            jax.ShapeDtypeStruct((N, 1), F32),
        ],
    )(x, W, di, do)


def _comb_body(a0_ref, a1_ref, h_ref, h0_ref, ri_ref, vi_ref, ro_ref,
               hn_ref, gn_ref):
    acc = a0_ref[...] + a1_ref[...]
    hn = 0.9 * (ri_ref[...] * acc + vi_ref[...] * h_ref[...]) \
        + 0.1 * h0_ref[...]
    hn_ref[...] = hn
    gn_ref[...] = hn * ro_ref[...]


def _comb_call(a0, a1, h, h0, ri, vi, ro):
    bsD = pl.BlockSpec((BR, D), lambda i: (i, 0))
    bs1 = pl.BlockSpec((BR, 1), lambda i: (i, 0))
    return pl.pallas_call(
        _comb_body,
        grid=(N // BR,),
        in_specs=[bsD, bsD, bsD, bsD, bs1, bs1, bs1],
        out_specs=[bsD, bsD],
        out_shape=[jax.ShapeDtypeStruct((N, D), F32)] * 2,
    )(a0, a1, h, h0, ri, vi, ro)


def _comb_gat_body(a0_ref, a1_ref, h_ref, h0_ref, ri_ref, vi_ref,
                   wg_ref, as_ref, an_ref, z_ref, zas_ref, zan_ref):
    acc = a0_ref[...] + a1_ref[...]
    hn = 0.9 * (ri_ref[...] * acc + vi_ref[...] * h_ref[...]) \
        + 0.1 * h0_ref[...]
    hs = 1.0 / (1.0 + jnp.exp(-hn))
    z = jnp.dot(hs, wg_ref[...], preferred_element_type=F32)
    z_ref[...] = z
    zas_ref[...] = jnp.dot(z, as_ref[...], preferred_element_type=F32)
    zan_ref[...] = jnp.dot(z, an_ref[...], preferred_element_type=F32)


def _comb_gat_call(a0, a1, h, h0, ri, vi, Wg, a_self, a_neigh):
    bsD = pl.BlockSpec((BR, D), lambda i: (i, 0))
    bs1 = pl.BlockSpec((BR, 1), lambda i: (i, 0))
    bsW = pl.BlockSpec((D, D), lambda i: (0, 0))
    bsv = pl.BlockSpec((D, 1), lambda i: (0, 0))
    return pl.pallas_call(
        _comb_gat_body,
        grid=(N // BR,),
        in_specs=[bsD, bsD, bsD, bsD, bs1, bs1, bsW, bsv, bsv],
        out_specs=[bsD, bs1, bs1],
        out_shape=[
            jax.ShapeDtypeStruct((N, D), F32),
            jax.ShapeDtypeStruct((N, 1), F32),
            jax.ShapeDtypeStruct((N, 1), F32),
        ],
    )(a0, a1, h, h0, ri, vi, Wg, a_self, a_neigh)


def _out_body(a0_ref, a1_ref, s0_ref, s1_ref, o_ref):
    s = s0_ref[...] + s1_ref[...]
    o = (a0_ref[...] + a1_ref[...]) / (s + 1e-9)
    o_ref[...] = jnp.where(o > 0, o, jnp.exp(o) - 1.0)


def _out_call(a0, a1, s0, s1):
    bsD = pl.BlockSpec((BR, D), lambda i: (i, 0))
    bs1 = pl.BlockSpec((BR, 1), lambda i: (i, 0))
    return pl.pallas_call(
        _out_body,
        grid=(N // BR,),
        in_specs=[bsD, bsD, bs1, bs1],
        out_specs=bsD,
        out_shape=jax.ShapeDtypeStruct((N, D), F32),
    )(a0, a1, s0, s1)


def kernel(x, edge_index, W_appnp, W_gat, a_self, a_neigh):
    src = edge_index[0]
    dst = edge_index[1]
    degin, degout = _deg_kernel(src, dst)
    h0, g, ri, vi, ro = _mm1_call(x, W_appnp, degin.reshape(NPAD, 1),
                                  degout.reshape(NPAD, 1))
    h = h0
    for _ in range(2):
        a0, a1 = _appnp_pass(g, src, dst)
        h, g = _comb_call(a0, a1, h, h0, ri, vi, ro)
    a0, a1 = _appnp_pass(g, src, dst)
    z, zas, zan = _comb_gat_call(a0, a1, h, h0, ri, vi, W_gat,
                                 a_self.reshape(D, 1), a_neigh.reshape(D, 1))
    az0, az1, s0, s1 = _gat_pass(z, zas.reshape(N), zan.reshape(N), src, dst)
    return _out_call(az0, az1, s0.reshape(NPAD, 1), s1.reshape(NPAD, 1))


# speedup vs baseline: 26.7729x; 1.7144x over previous
import functools

import jax
import jax.numpy as jnp
from jax import lax
from jax.experimental import pallas as pl
from jax.experimental.pallas import tpu as pltpu
from jax.experimental.pallas import tpu_sc as plsc

N = 10000
E = 320000
D = 128
NC = 2
NS = 16
NW = NC * NS
NPAD = 10240
RPS = NPAD // NS
EB = 128
NU = E // EB
UW = NU // NW
UT = NU // NS
F32 = jnp.float32

_MESH = plsc.VectorSubcoreMesh(core_axis_name="c", subcore_axis_name="s")
_SC_PARAMS = pltpu.CompilerParams(needs_layout_passes=False)

_Z16 = lambda: jnp.zeros((16,), F32)


def _zero_shared_rows(ztile_v, sh_ref, sid):
    for b in range(RPS // 16):
        pltpu.sync_copy(ztile_v, sh_ref.at[pl.ds(sid * RPS + b * 16, 16)])


def _dump_shared_rows(sh_ref, out_ref, sid):
    pltpu.sync_copy(sh_ref.at[pl.ds(sid * RPS, RPS)],
                    out_ref.at[pl.ds(sid * RPS, RPS)])


def _fill_ztile(ztile_v):
    for r in range(16):
        for k in range(8):
            ztile_v[r, pl.ds(k * 16, 16)] = _Z16()


@functools.partial(
    pl.kernel,
    out_type=(jax.ShapeDtypeStruct((NPAD,), F32),
              jax.ShapeDtypeStruct((NPAD,), F32)),
    mesh=_MESH,
    compiler_params=_SC_PARAMS,
    scratch_types=[
        pltpu.VMEM((EB,), jnp.int32),
        pltpu.VMEM((EB,), jnp.int32),
        pltpu.VMEM((EB,), F32),
        pltpu.VMEM((16,), F32),
        pltpu.VMEM_SHARED((NPAD,), F32),
        pltpu.SemaphoreType.DMA,
        pltpu.SemaphoreType.DMA,
    ],
)
def _deg_kernel(src_hbm, dst_hbm, degin_hbm, degout_hbm,
                idx0_v, idx1_v, ones_v, z16_v, deg_sh, semi0, semi1):
    cid = lax.axis_index("c")
    sid = lax.axis_index("s")
    z16_v[...] = _Z16()
    for k in range(EB // 16):
        ones_v[pl.ds(k * 16, 16)] = jnp.ones((16,), F32)
    for b in range(RPS // 16):
        pltpu.sync_copy(z16_v, deg_sh.at[pl.ds(sid * RPS + b * 16, 16)])
    plsc.subcore_barrier()

    nu = UT + jnp.where(sid < NU - UT * NS, 1, 0)

    def tally(edge_hbm):
        bufs = ((idx0_v, semi0), (idx1_v, semi1))
        pltpu.sync_copy(edge_hbm.at[pl.ds(sid * EB, EB)], idx0_v)

        def pair(gp, _):
            for b in range(2):
                u = gp * 2 + b
                idx_b, semi_b = bufs[b]
                idx_n, semi_n = bufs[1 - b]

                @pl.when(u < nu)
                def _():
                    @pl.when(u + 1 < nu)
                    def _():
                        base = ((u + 1) * NS + sid) * EB
                        pltpu.async_copy(edge_hbm.at[pl.ds(base, EB)],
                                         idx_n, semi_n)

                    @pl.when(u > 0)
                    def _():
                        pltpu.make_async_copy(
                            edge_hbm.at[pl.ds(0, EB)], idx_b, semi_b).wait()

                    pltpu.sync_copy(ones_v, deg_sh.at[idx_b], add=True)
            return 0

        lax.fori_loop(0, (UT + 2) // 2, pair, 0)

    @pl.when(cid == 0)
    def _():
        tally(dst_hbm)

    @pl.when(cid == 1)
    def _():
        tally(src_hbm)

    plsc.subcore_barrier()

    @pl.when(cid == 0)
    def _():
        _dump_shared_rows(deg_sh, degin_hbm, sid)

    @pl.when(cid == 1)
    def _():
        _dump_shared_rows(deg_sh, degout_hbm, sid)


@functools.partial(
    pl.kernel,
    out_type=(jax.ShapeDtypeStruct((NPAD, D), F32),
              jax.ShapeDtypeStruct((NPAD, D), F32)),
    mesh=_MESH,
    compiler_params=_SC_PARAMS,
    scratch_types=[
        pltpu.VMEM((EB,), jnp.int32),
        pltpu.VMEM((EB,), jnp.int32),
        pltpu.VMEM((EB,), jnp.int32),
        pltpu.VMEM((EB,), jnp.int32),
        pltpu.VMEM((EB, D), F32),
        pltpu.VMEM((EB, D), F32),
        pltpu.VMEM((16, 128), F32),
        pltpu.VMEM_SHARED((NPAD, D), F32),
        pltpu.SemaphoreType.DMA,
        pltpu.SemaphoreType.DMA,
        pltpu.SemaphoreType.DMA,
        pltpu.SemaphoreType.DMA,
    ],
)
def _appnp_pass(g_hbm, src_hbm, dst_hbm, acc0_hbm, acc1_hbm,
                sidx0_v, sidx1_v, didx0_v, didx1_v, rows0_v, rows1_v,
                ztile_v, acc_sh, semi0, semi1, semg0, semg1):
    cid = lax.axis_index("c")
    sid = lax.axis_index("s")
    wid = cid * NS + sid
    _fill_ztile(ztile_v)
    _zero_shared_rows(ztile_v, acc_sh, sid)
    plsc.subcore_barrier()

    nu = UW + jnp.where(wid < NU - UW * NW, 1, 0)

    def ebase(u):
        return (u * NW + wid) * EB

    bufs = ((sidx0_v, didx0_v, rows0_v, semi0, semg0),
            (sidx1_v, didx1_v, rows1_v, semi1, semg1))

    pltpu.sync_copy(src_hbm.at[pl.ds(wid * EB, EB)], sidx0_v)
    pltpu.sync_copy(dst_hbm.at[pl.ds(wid * EB, EB)], didx0_v)
    pltpu.async_copy(src_hbm.at[pl.ds((NW + wid) * EB, EB)], sidx1_v,
                     semi1)
    pltpu.async_copy(dst_hbm.at[pl.ds((NW + wid) * EB, EB)], didx1_v,
                     semi1)
    pltpu.async_copy(g_hbm.at[sidx0_v], rows0_v, semg0)

    def pair(gp, _):
        for b in range(2):
            u = gp * 2 + b
            sidx_b, didx_b, rows_b, semi_b, semg_b = bufs[b]
            sidx_n, didx_n, rows_n, semi_n, semg_n = bufs[1 - b]

            @pl.when(u < nu)
            def _():
                pltpu.make_async_copy(g_hbm.at[sidx_b], rows_b,
                                      semg_b).wait()

                @pl.when(u + 1 < nu)
                def _():
                    pltpu.make_async_copy(src_hbm.at[pl.ds(0, EB)], sidx_n,
                                          semi_n).wait()
                    pltpu.make_async_copy(dst_hbm.at[pl.ds(0, EB)], didx_n,
                                          semi_n).wait()
                    pltpu.async_copy(g_hbm.at[sidx_n], rows_n, semg_n)

                pltpu.sync_copy(rows_b, acc_sh.at[didx_b], add=True)

                @pl.when(u + 2 < nu)
                def _():
                    b2 = ebase(u + 2)
                    pltpu.async_copy(src_hbm.at[pl.ds(b2, EB)], sidx_b,
                                     semi_b)
                    pltpu.async_copy(dst_hbm.at[pl.ds(b2, EB)], didx_b,
                                     semi_b)
        return 0

    lax.fori_loop(0, (UW + 2) // 2, pair, 0)
    plsc.subcore_barrier()

    @pl.when(cid == 0)
    def _():
        _dump_shared_rows(acc_sh, acc0_hbm, sid)

    @pl.when(cid == 1)
    def _():
        _dump_shared_rows(acc_sh, acc1_hbm, sid)


@functools.partial(
    pl.kernel,
    out_type=(jax.ShapeDtypeStruct((NPAD, D), F32),
              jax.ShapeDtypeStruct((NPAD, D), F32),
              jax.ShapeDtypeStruct((NPAD,), F32),
              jax.ShapeDtypeStruct((NPAD,), F32)),
    mesh=_MESH,
    compiler_params=_SC_PARAMS,
    scratch_types=[
        pltpu.VMEM((EB,), jnp.int32),
        pltpu.VMEM((EB,), jnp.int32),
        pltpu.VMEM((EB,), jnp.int32),
        pltpu.VMEM((EB,), jnp.int32),
        pltpu.VMEM((EB, D), F32),
        pltpu.VMEM((EB, D), F32),
        pltpu.VMEM((EB,), F32),
        pltpu.VMEM((EB,), F32),
        pltpu.VMEM((EB,), F32),
        pltpu.VMEM((EB,), F32),
        pltpu.VMEM((EB,), F32),
        pltpu.VMEM((EB,), F32),
        pltpu.VMEM((16, 128), F32),
        pltpu.VMEM_SHARED((NPAD, D), F32),
        pltpu.VMEM_SHARED((NPAD,), F32),
        pltpu.SemaphoreType.DMA,
        pltpu.SemaphoreType.DMA,
        pltpu.SemaphoreType.DMA,
        pltpu.SemaphoreType.DMA,
    ],
)
def _gat_pass(z_hbm, asf_hbm, anb_hbm, src_hbm, dst_hbm,
              accz0_hbm, accz1_hbm, s0_hbm, s1_hbm,
              sidx0_v, sidx1_v, didx0_v, didx1_v, rows0_v, rows1_v,
              ee0_v, ee1_v, as0_v, as1_v, an0_v, an1_v, ztile_v,
              accz_sh, s_sh, semi0, semi1, semg0, semg1):
    cid = lax.axis_index("c")
    sid = lax.axis_index("s")
    wid = cid * NS + sid
    _fill_ztile(ztile_v)
    _zero_shared_rows(ztile_v, accz_sh, sid)
    for b in range(RPS // 128):
        pltpu.sync_copy(ztile_v.at[0],
                        s_sh.at[pl.ds(sid * RPS + b * 128, 128)])
    plsc.subcore_barrier()

    nu = UW + jnp.where(wid < NU - UW * NW, 1, 0)

    def ebase(u):
        return (u * NW + wid) * EB

    bufs = ((sidx0_v, didx0_v, rows0_v, ee0_v, as0_v, an0_v, semi0, semg0),
            (sidx1_v, didx1_v, rows1_v, ee1_v, as1_v, an1_v, semi1, semg1))

    def launch_gathers(sidx_v, didx_v, rows_v, as_v, an_v, semg):
        pltpu.async_copy(z_hbm.at[sidx_v], rows_v, semg)
        pltpu.async_copy(asf_hbm.at[didx_v], as_v, semg)
        pltpu.async_copy(anb_hbm.at[sidx_v], an_v, semg)

    def wait_gathers(sidx_v, didx_v, rows_v, as_v, an_v, semg):
        pltpu.make_async_copy(z_hbm.at[sidx_v], rows_v, semg).wait()
        pltpu.make_async_copy(asf_hbm.at[didx_v], as_v, semg).wait()
        pltpu.make_async_copy(anb_hbm.at[sidx_v], an_v, semg).wait()

    pltpu.sync_copy(src_hbm.at[pl.ds(wid * EB, EB)], sidx0_v)
    pltpu.sync_copy(dst_hbm.at[pl.ds(wid * EB, EB)], didx0_v)
    pltpu.async_copy(src_hbm.at[pl.ds((NW + wid) * EB, EB)], sidx1_v,
                     semi1)
    pltpu.async_copy(dst_hbm.at[pl.ds((NW + wid) * EB, EB)], didx1_v,
                     semi1)
    launch_gathers(sidx0_v, didx0_v, rows0_v, as0_v, an0_v, semg0)

    def pair(gp, _):
        for b in range(2):
            u = gp * 2 + b
            sidx_b, didx_b, rows_b, ee_b, as_b, an_b, semi_b, semg_b = bufs[b]
            (sidx_n, didx_n, rows_n, _ee_n, as_n, an_n, semi_n,
             semg_n) = bufs[1 - b]

            @pl.when(u < nu)
            def _():
                @pl.when(u + 1 < nu)
                def _():
                    pltpu.make_async_copy(src_hbm.at[pl.ds(0, EB)], sidx_n,
                                          semi_n).wait()
                    pltpu.make_async_copy(dst_hbm.at[pl.ds(0, EB)], didx_n,
                                          semi_n).wait()
                    launch_gathers(sidx_n, didx_n, rows_n, as_n, an_n,
                                   semg_n)

                wait_gathers(sidx_b, didx_b, rows_b, as_b, an_b, semg_b)

                for k in range(EB // 16):
                    sl = pl.ds(k * 16, 16)
                    lg = as_b[sl] + an_b[sl]
                    lg = jnp.where(lg > 0, lg, 0.2 * lg)
                    ee_b[sl] = jnp.exp(lg)

                def scale_16rows(t, _):
                    e16 = ee_b[pl.ds(t * 16, 16)]
                    for j in range(16):
                        es = e16[j]
                        row = t * 16 + j
                        for k in range(D // 16):
                            sl = pl.ds(k * 16, 16)
                            rows_b[row, sl] = rows_b[row, sl] * es
                    return 0

                lax.fori_loop(0, EB // 16, scale_16rows, 0)
                pltpu.sync_copy(rows_b, accz_sh.at[didx_b], add=True)
                pltpu.sync_copy(ee_b, s_sh.at[didx_b], add=True)

                @pl.when(u + 2 < nu)
                def _():
                    b2 = ebase(u + 2)
                    pltpu.async_copy(src_hbm.at[pl.ds(b2, EB)], sidx_b,
                                     semi_b)
                    pltpu.async_copy(dst_hbm.at[pl.ds(b2, EB)], didx_b,
                                     semi_b)
        return 0

    lax.fori_loop(0, (UW + 2) // 2, pair, 0)
    plsc.subcore_barrier()

    @pl.when(cid == 0)
    def _():
        _dump_shared_rows(accz_sh, accz0_hbm, sid)
        _dump_shared_rows(s_sh, s0_hbm, sid)

    @pl.when(cid == 1)
    def _():
        _dump_shared_rows(accz_sh, accz1_hbm, sid)
        _dump_shared_rows(s_sh, s1_hbm, sid)


BR = 2000


def _mm1_body(x_ref, w_ref, di_ref, do_ref,
              h0_ref, g0_ref, ri_ref, vi_ref, ro_ref):
    h0 = jnp.dot(x_ref[...], w_ref[...], preferred_element_type=F32)
    din = di_ref[...] + 1.0
    dout = do_ref[...] + 1.0
    ri = lax.rsqrt(din)
    ro = lax.rsqrt(dout)
    h0_ref[...] = h0
    g0_ref[...] = h0 * ro
    ri_ref[...] = ri
    vi_ref[...] = 1.0 / din
    ro_ref[...] = ro


def _mm1_call(x, W, di, do):
    return pl.pallas_call(
        _mm1_body,
        grid=(N // BR,),
        in_specs=[
            pl.BlockSpec((BR, D), lambda i: (i, 0)),
            pl.BlockSpec((D, D), lambda i: (0, 0)),
            pl.BlockSpec((BR, 1), lambda i: (i, 0)),
            pl.BlockSpec((BR, 1), lambda i: (i, 0)),
        ],
        out_specs=[
            pl.BlockSpec((BR, D), lambda i: (i, 0)),
            pl.BlockSpec((BR, D), lambda i: (i, 0)),
            pl.BlockSpec((BR, 1), lambda i: (i, 0)),
            pl.BlockSpec((BR, 1), lambda i: (i, 0)),
            pl.BlockSpec((BR, 1), lambda i: (i, 0)),
        ],
        out_shape=[
            jax.ShapeDtypeStruct((N, D), F32),
            jax.ShapeDtypeStruct((N, D), F32),
            jax.ShapeDtypeStruct((N, 1), F32),
            jax.ShapeDtypeStruct((N, 1), F32),
            jax.ShapeDtypeStruct((N, 1), F32),
        ],
    )(x, W, di, do)


def _comb_body(a0_ref, a1_ref, h_ref, h0_ref, ri_ref, vi_ref, ro_ref,
               hn_ref, gn_ref):
    acc = a0_ref[...] + a1_ref[...]
    hn = 0.9 * (ri_ref[...] * acc + vi_ref[...] * h_ref[...]) \
        + 0.1 * h0_ref[...]
    hn_ref[...] = hn
    gn_ref[...] = hn * ro_ref[...]


def _comb_call(a0, a1, h, h0, ri, vi, ro):
    bsD = pl.BlockSpec((BR, D), lambda i: (i, 0))
    bs1 = pl.BlockSpec((BR, 1), lambda i: (i, 0))
    return pl.pallas_call(
        _comb_body,
        grid=(N // BR,),
        in_specs=[bsD, bsD, bsD, bsD, bs1, bs1, bs1],
        out_specs=[bsD, bsD],
        out_shape=[jax.ShapeDtypeStruct((N, D), F32)] * 2,
    )(a0, a1, h, h0, ri, vi, ro)


def _comb_gat_body(a0_ref, a1_ref, h_ref, h0_ref, ri_ref, vi_ref,
                   wg_ref, as_ref, an_ref, z_ref, zas_ref, zan_ref):
    acc = a0_ref[...] + a1_ref[...]
    hn = 0.9 * (ri_ref[...] * acc + vi_ref[...] * h_ref[...]) \
        + 0.1 * h0_ref[...]
    hs = 1.0 / (1.0 + jnp.exp(-hn))
    z = jnp.dot(hs, wg_ref[...], preferred_element_type=F32)
    z_ref[...] = z
    zas_ref[...] = jnp.dot(z, as_ref[...], preferred_element_type=F32)
    zan_ref[...] = jnp.dot(z, an_ref[...], preferred_element_type=F32)


def _comb_gat_call(a0, a1, h, h0, ri, vi, Wg, a_self, a_neigh):
    bsD = pl.BlockSpec((BR, D), lambda i: (i, 0))
    bs1 = pl.BlockSpec((BR, 1), lambda i: (i, 0))
    bsW = pl.BlockSpec((D, D), lambda i: (0, 0))
    bsv = pl.BlockSpec((D, 1), lambda i: (0, 0))
    return pl.pallas_call(
        _comb_gat_body,
        grid=(N // BR,),
        in_specs=[bsD, bsD, bsD, bsD, bs1, bs1, bsW, bsv, bsv],
        out_specs=[bsD, bs1, bs1],
        out_shape=[
            jax.ShapeDtypeStruct((N, D), F32),
            jax.ShapeDtypeStruct((N, 1), F32),
            jax.ShapeDtypeStruct((N, 1), F32),
        ],
    )(a0, a1, h, h0, ri, vi, Wg, a_self, a_neigh)


def _out_body(a0_ref, a1_ref, s0_ref, s1_ref, o_ref):
    s = s0_ref[...] + s1_ref[...]
    o = (a0_ref[...] + a1_ref[...]) / (s + 1e-9)
    o_ref[...] = jnp.where(o > 0, o, jnp.exp(o) - 1.0)


def _out_call(a0, a1, s0, s1):
    bsD = pl.BlockSpec((BR, D), lambda i: (i, 0))
    bs1 = pl.BlockSpec((BR, 1), lambda i: (i, 0))
    return pl.pallas_call(
        _out_body,
        grid=(N // BR,),
        in_specs=[bsD, bsD, bs1, bs1],
        out_specs=bsD,
        out_shape=jax.ShapeDtypeStruct((N, D), F32),
    )(a0, a1, s0, s1)


def kernel(x, edge_index, W_appnp, W_gat, a_self, a_neigh):
    src = edge_index[0]
    dst = edge_index[1]
    degin, degout = _deg_kernel(src, dst)
    h0, g, ri, vi, ro = _mm1_call(x, W_appnp, degin.reshape(NPAD, 1),
                                  degout.reshape(NPAD, 1))
    h = h0
    for _ in range(2):
        a0, a1 = _appnp_pass(g, src, dst)
        h, g = _comb_call(a0, a1, h, h0, ri, vi, ro)
    a0, a1 = _appnp_pass(g, src, dst)
    z, zas, zan = _comb_gat_call(a0, a1, h, h0, ri, vi, W_gat,
                                 a_self.reshape(D, 1), a_neigh.reshape(D, 1))
    az0, az1, s0, s1 = _gat_pass(z, zas.reshape(N), zan.reshape(N), src, dst)
    return _out_call(az0, az1, s0.reshape(NPAD, 1), s1.reshape(NPAD, 1))
